# Initial kernel scaffold; baseline (speedup 1.0000x reference)
import jax
import jax.numpy as jnp
from jax.experimental import pallas as pl


def kernel(x, edge_index, W1, b1, W2, b2):
    raise NotImplementedError("write your pallas kernel here")


# speedup vs baseline: 13.2359x; 13.2359x over previous
import functools
import math

import jax
import jax.numpy as jnp
from jax import lax
from jax.experimental import pallas as pl
from jax.experimental.pallas import tpu as pltpu
from jax.experimental.pallas import tpu_sc as plsc

NC = 2
NS = 16
NW = NC * NS
LANES = 16
CHUNK = 128


def _sc_mesh():
    return plsc.VectorSubcoreMesh(
        core_axis_name="c", subcore_axis_name="s", num_cores=NC, num_subcores=NS
    )


def _sc_degree(dst_r, np_rows):
    nch = dst_r.shape[1]
    rpt = np_rows // NS

    @functools.partial(
        pl.kernel,
        out_type=jax.ShapeDtypeStruct((NC, np_rows), jnp.float32),
        mesh=_sc_mesh(),
        scratch_types=[
            pltpu.VMEM((nch, CHUNK), jnp.int32),
            pltpu.VMEM((CHUNK,), jnp.float32),
            pltpu.VMEM((rpt,), jnp.float32),
            pltpu.VMEM_SHARED((np_rows,), jnp.float32),
        ],
    )
    def k(dst_hbm, out_hbm, dst_v, ones_v, z_v, acc_sh):
        c = lax.axis_index("c")
        s = lax.axis_index("s")
        wid = c * NS + s
        pltpu.sync_copy(dst_hbm.at[wid], dst_v)
        for i in range(CHUNK // LANES):
            ones_v[pl.ds(i * LANES, LANES)] = jnp.full((LANES,), 1.0, jnp.float32)
        for i in range(rpt // LANES):
            z_v[pl.ds(i * LANES, LANES)] = jnp.zeros((LANES,), jnp.float32)
        pltpu.sync_copy(z_v, acc_sh.at[pl.ds(s * rpt, rpt)])
        plsc.subcore_barrier()

        def step(j, carry):
            pltpu.sync_copy(ones_v, acc_sh.at[dst_v.at[j]], add=True)
            return carry

        lax.fori_loop(0, nch, step, 0)
        plsc.subcore_barrier()
        pltpu.sync_copy(acc_sh.at[pl.ds(s * rpt, rpt)], out_hbm.at[c, pl.ds(s * rpt, rpt)])

    return k(dst_r)


def _sc_aggregate(table, src_r, dst_r, np_rows, d):
    nch = src_r.shape[1]
    rpt = np_rows // NS

    @functools.partial(
        pl.kernel,
        out_type=jax.ShapeDtypeStruct((NC, np_rows, d), jnp.float32),
        mesh=_sc_mesh(),
        scratch_types=[
            pltpu.VMEM((nch, CHUNK), jnp.int32),
            pltpu.VMEM((nch, CHUNK), jnp.int32),
            pltpu.VMEM((CHUNK, d), jnp.float32),
            pltpu.VMEM_SHARED((np_rows, d), jnp.float32),
            pltpu.SemaphoreType.DMA,
        ],
    )
    def k(tab_hbm, src_hbm, dst_hbm, out_hbm, src_v, dst_v, rows_v, acc_sh, sem):
        c = lax.axis_index("c")
        s = lax.axis_index("s")
        wid = c * NS + s
        pltpu.sync_copy(src_hbm.at[wid], src_v)
        pltpu.sync_copy(dst_hbm.at[wid], dst_v)
        pltpu.sync_copy(
            tab_hbm.at[pl.ds(s * rpt, rpt)], acc_sh.at[pl.ds(s * rpt, rpt)]
        )
        plsc.subcore_barrier()

        def step(j, carry):
            pltpu.async_copy(tab_hbm.at[src_v.at[j]], rows_v, sem).wait()
            pltpu.sync_copy(rows_v, acc_sh.at[dst_v.at[j]], add=True)
            return carry

        lax.fori_loop(0, nch, step, 0)
        plsc.subcore_barrier()
        pltpu.sync_copy(
            acc_sh.at[pl.ds(s * rpt, rpt)], out_hbm.at[c, pl.ds(s * rpt, rpt)]
        )

    return k(table, src_r, dst_r)


def _tc_matmul(xp, w):
    np_rows = xp.shape[0]
    h = w.shape[0]

    def body(x_ref, w_ref, o_ref):
        o_ref[...] = lax.dot_general(
            x_ref[...], w_ref[...], (((1,), (1,)), ((), ())),
            preferred_element_type=jnp.float32,
        )

    return pl.pallas_call(
        body, out_shape=jax.ShapeDtypeStruct((np_rows, h), jnp.float32)
    )(xp, w)


def _tc_scale(dparts, hraw):
    np_rows, h = hraw.shape

    def body(d_ref, h_ref, hs_ref, dis_ref):
        deg = d_ref[0] + d_ref[1] + 1.0
        dis = lax.rsqrt(deg)
        dis_ref[...] = dis
        hs_ref[...] = h_ref[...] * dis

    return pl.pallas_call(
        body,
        out_shape=[
            jax.ShapeDtypeStruct((np_rows, h), jnp.float32),
            jax.ShapeDtypeStruct((np_rows, 1), jnp.float32),
        ],
    )(dparts, hraw)


def _tc_mid(parts, hs1, dis, b1, w2):
    np_rows = hs1.shape[0]
    o = w2.shape[0]

    def body(p_ref, hs1_ref, dis_ref, b1_ref, w2_ref, hs2_ref):
        agg = p_ref[0] + p_ref[1] - hs1_ref[...]
        t = jnp.maximum(agg * dis_ref[...] + b1_ref[...], 0.0)
        h2 = lax.dot_general(
            t, w2_ref[...], (((1,), (1,)), ((), ())),
            preferred_element_type=jnp.float32,
        )
        hs2_ref[...] = h2 * dis_ref[...]

    return pl.pallas_call(
        body, out_shape=jax.ShapeDtypeStruct((np_rows, o), jnp.float32)
    )(parts, hs1, dis, b1, w2)


def _tc_final(parts, hs2, dis, b2):
    np_rows = hs2.shape[0]
    o = b2.shape[1]

    def body(q_ref, hs2_ref, dis_ref, b2_ref, o_ref):
        agg = q_ref[0] + q_ref[1] - hs2_ref[...]
        u = (agg * dis_ref[...])[:, :o] + b2_ref[...]
        m = jnp.max(u, axis=1, keepdims=True)
        e = jnp.exp(u - m)
        lse = jnp.log(jnp.sum(e, axis=1, keepdims=True)) + m
        o_ref[...] = u - lse

    return pl.pallas_call(
        body, out_shape=jax.ShapeDtypeStruct((np_rows, o), jnp.float32)
    )(parts, hs2, dis, b2)


@jax.jit
def kernel(x, edge_index, W1, b1, W2, b2):
    n, _ = x.shape
    e = edge_index.shape[1]

    np_rows = (NS * LANES) * math.ceil((n + 1) / (NS * LANES))
    dummy = n

    ept = CHUNK * math.ceil(e / (NW * CHUNK))
    etot = ept * NW
    nch = ept // CHUNK
    src = edge_index[0].astype(jnp.int32)
    dst = edge_index[1].astype(jnp.int32)
    src_r = jnp.concatenate([src, jnp.zeros((etot - e,), jnp.int32)]).reshape(
        NW, nch, CHUNK
    )
    dst_r = jnp.concatenate([dst, jnp.full((etot - e,), dummy, jnp.int32)]).reshape(
        NW, nch, CHUNK
    )

    x_pad = jnp.pad(x, ((0, np_rows - n), (0, 0)))

    o = W2.shape[0]
    w2p = jnp.pad(W2, ((0, 128 - o), (0, 0)))

    hraw = _tc_matmul(x_pad, W1)
    dparts = _sc_degree(dst_r, np_rows)
    hs1, dis = _tc_scale(dparts.reshape(NC, np_rows, 1), hraw)
    p = _sc_aggregate(hs1, src_r, dst_r, np_rows, hs1.shape[1])
    hs2 = _tc_mid(p, hs1, dis, b1.reshape(1, -1), w2p)
    q = _sc_aggregate(hs2, src_r, dst_r, np_rows, hs2.shape[1])
    out = _tc_final(q, hs2, dis, b2.reshape(1, -1))
    return out[:n]


# speedup vs baseline: 13.7216x; 1.0367x over previous
import functools
import math

import jax
import jax.numpy as jnp
from jax import lax
from jax.experimental import pallas as pl
from jax.experimental.pallas import tpu as pltpu
from jax.experimental.pallas import tpu_sc as plsc

NC = 2
NS = 16
NW = NC * NS
LANES = 16
CHUNK = 128


def _sc_mesh():
    return plsc.VectorSubcoreMesh(
        core_axis_name="c", subcore_axis_name="s", num_cores=NC, num_subcores=NS
    )


def _sc_degree(dst_r, np_rows):
    nch = dst_r.shape[1]
    rpt = np_rows // NS

    @functools.partial(
        pl.kernel,
        out_type=jax.ShapeDtypeStruct((NC, np_rows), jnp.float32),
        mesh=_sc_mesh(),
        scratch_types=[
            pltpu.VMEM((nch, CHUNK), jnp.int32),
            pltpu.VMEM((CHUNK,), jnp.float32),
            pltpu.VMEM((rpt,), jnp.float32),
            pltpu.VMEM_SHARED((np_rows,), jnp.float32),
        ],
    )
    def k(dst_hbm, out_hbm, dst_v, ones_v, z_v, acc_sh):
        c = lax.axis_index("c")
        s = lax.axis_index("s")
        wid = c * NS + s
        pltpu.sync_copy(dst_hbm.at[wid], dst_v)
        for i in range(CHUNK // LANES):
            ones_v[pl.ds(i * LANES, LANES)] = jnp.full((LANES,), 1.0, jnp.float32)
        for i in range(rpt // LANES):
            z_v[pl.ds(i * LANES, LANES)] = jnp.zeros((LANES,), jnp.float32)
        pltpu.sync_copy(z_v, acc_sh.at[pl.ds(s * rpt, rpt)])
        plsc.subcore_barrier()

        def step(j, carry):
            pltpu.sync_copy(ones_v, acc_sh.at[dst_v.at[j]], add=True)
            return carry

        lax.fori_loop(0, nch, step, 0)
        plsc.subcore_barrier()
        pltpu.sync_copy(acc_sh.at[pl.ds(s * rpt, rpt)], out_hbm.at[c, pl.ds(s * rpt, rpt)])

    return k(dst_r)


def _sc_aggregate(table, src_r, dst_r, np_rows, d, tc_tiling=True):
    nch = src_r.shape[1]
    assert nch % 4 == 0 and nch >= 8
    nhalf = nch // 2
    assert nhalf % 2 == 0
    rpt = np_rows // NS

    @functools.partial(
        pl.kernel,
        out_type=jax.ShapeDtypeStruct((NC, np_rows, d), jnp.float32),
        mesh=_sc_mesh(),
        compiler_params=pltpu.CompilerParams(use_tc_tiling_on_sc=tc_tiling),
        scratch_types=[
            pltpu.VMEM((nhalf, CHUNK), jnp.int32),
            pltpu.VMEM((nhalf, CHUNK), jnp.int32),
            pltpu.VMEM((CHUNK, d), jnp.float32),
            pltpu.VMEM((CHUNK, d), jnp.float32),
            pltpu.VMEM_SHARED((np_rows, d), jnp.float32),
            pltpu.SemaphoreType.DMA,
        ],
    )
    def k(tab_hbm, src_hbm, dst_hbm, out_hbm, src_v, dst_v, r0, r1, acc_sh, gsem):
        c = lax.axis_index("c")
        s = lax.axis_index("s")
        wid = c * NS + s
        pltpu.sync_copy(
            tab_hbm.at[pl.ds(s * rpt, rpt)], acc_sh.at[pl.ds(s * rpt, rpt)]
        )
        plsc.subcore_barrier()

        bufs = [r0, r1]
        for half in range(2):
            pltpu.sync_copy(src_hbm.at[wid, pl.ds(half * nhalf, nhalf)], src_v)
            pltpu.sync_copy(dst_hbm.at[wid, pl.ds(half * nhalf, nhalf)], dst_v)
            pltpu.async_copy(tab_hbm.at[src_v.at[0]], bufs[0], gsem)

            def body(j2, carry):
                j = j2 * 2
                pltpu.async_copy(tab_hbm.at[src_v.at[j + 1]], bufs[1], gsem)
                pltpu.make_async_copy(
                    tab_hbm.at[pl.ds(0, CHUNK)], bufs[0], gsem
                ).wait()
                pltpu.sync_copy(bufs[0], acc_sh.at[dst_v.at[j]], add=True)

                @pl.when(j2 < nhalf // 2 - 1)
                def _f():
                    pltpu.async_copy(tab_hbm.at[src_v.at[j + 2]], bufs[0], gsem)

                pltpu.make_async_copy(
                    tab_hbm.at[pl.ds(0, CHUNK)], bufs[1], gsem
                ).wait()
                pltpu.sync_copy(bufs[1], acc_sh.at[dst_v.at[j + 1]], add=True)
                return carry

            lax.fori_loop(0, nhalf // 2, body, 0)

        plsc.subcore_barrier()
        pltpu.sync_copy(
            acc_sh.at[pl.ds(s * rpt, rpt)], out_hbm.at[c, pl.ds(s * rpt, rpt)]
        )

    return k(table, src_r, dst_r)


def _tc_matmul(xp, w):
    np_rows = xp.shape[0]
    h = w.shape[0]

    def body(x_ref, w_ref, o_ref):
        o_ref[...] = lax.dot_general(
            x_ref[...], w_ref[...], (((1,), (1,)), ((), ())),
            preferred_element_type=jnp.float32,
        )

    return pl.pallas_call(
        body, out_shape=jax.ShapeDtypeStruct((np_rows, h), jnp.float32)
    )(xp, w)


def _tc_scale(dparts, hraw):
    np_rows, h = hraw.shape

    def body(d_ref, h_ref, hs_ref, dis_ref):
        deg = d_ref[0] + d_ref[1] + 1.0
        dis = lax.rsqrt(deg)
        dis_ref[...] = dis
        hs_ref[...] = h_ref[...] * dis

    return pl.pallas_call(
        body,
        out_shape=[
            jax.ShapeDtypeStruct((np_rows, h), jnp.float32),
            jax.ShapeDtypeStruct((np_rows, 1), jnp.float32),
        ],
    )(dparts, hraw)


def _tc_mid(parts, hs1, dis, b1, w2):
    np_rows = hs1.shape[0]
    o = w2.shape[0]

    def body(p_ref, hs1_ref, dis_ref, b1_ref, w2_ref, hs2_ref):
        agg = p_ref[0] + p_ref[1] - hs1_ref[...]
        t = jnp.maximum(agg * dis_ref[...] + b1_ref[...], 0.0)
        h2 = lax.dot_general(
            t, w2_ref[...], (((1,), (1,)), ((), ())),
            preferred_element_type=jnp.float32,
        )
        hs2_ref[...] = h2 * dis_ref[...]

    return pl.pallas_call(
        body, out_shape=jax.ShapeDtypeStruct((np_rows, o), jnp.float32)
    )(parts, hs1, dis, b1, w2)


def _tc_final(parts, hs2, dis, b2):
    np_rows = hs2.shape[0]
    o = b2.shape[1]

    def body(q_ref, hs2_ref, dis_ref, b2_ref, o_ref):
        agg = q_ref[0] + q_ref[1] - hs2_ref[...]
        u = (agg * dis_ref[...])[:, :o] + b2_ref[...]
        m = jnp.max(u, axis=1, keepdims=True)
        e = jnp.exp(u - m)
        lse = jnp.log(jnp.sum(e, axis=1, keepdims=True)) + m
        o_ref[...] = u - lse

    return pl.pallas_call(
        body, out_shape=jax.ShapeDtypeStruct((np_rows, o), jnp.float32)
    )(parts, hs2, dis, b2)


@jax.jit
def kernel(x, edge_index, W1, b1, W2, b2):
    n, _ = x.shape
    e = edge_index.shape[1]

    np_rows = (NS * LANES) * math.ceil((n + 1) / (NS * LANES))
    dummy = n

    nch = 4 * math.ceil(e / (NW * CHUNK * 4))
    ept = CHUNK * nch
    etot = ept * NW
    src = edge_index[0].astype(jnp.int32)
    dst = edge_index[1].astype(jnp.int32)
    src_r = jnp.concatenate([src, jnp.zeros((etot - e,), jnp.int32)]).reshape(
        NW, nch, CHUNK
    )
    dst_r = jnp.concatenate([dst, jnp.full((etot - e,), dummy, jnp.int32)]).reshape(
        NW, nch, CHUNK
    )

    x_pad = jnp.pad(x, ((0, np_rows - n), (0, 0)))

    hraw = _tc_matmul(x_pad, W1)
    dparts = _sc_degree(dst_r, np_rows)
    hs1, dis = _tc_scale(dparts.reshape(NC, np_rows, 1), hraw)
    p = _sc_aggregate(hs1, src_r, dst_r, np_rows, hs1.shape[1])
    hs2 = _tc_mid(p, hs1, dis, b1.reshape(1, -1), W2)
    q = _sc_aggregate(hs2, src_r, dst_r, np_rows, hs2.shape[1], tc_tiling=False)
    out = _tc_final(q, hs2, dis, b2.reshape(1, -1))
    return out[:n]
